# Initial kernel scaffold; baseline (speedup 1.0000x reference)
import jax
import jax.numpy as jnp
from jax.experimental import pallas as pl


def kernel(x, edge_index, W1_0, b1_0, W1_1, b1_1, W1_2, b1_2, W2_0, b2_0, W2_1, b2_1, W2_2, b2_2, Wf, bf):
    raise NotImplementedError("write your pallas kernel here")


# speedup vs baseline: 9.7404x; 9.7404x over previous
import functools

import jax
import jax.numpy as jnp
from jax import lax
from jax.experimental import pallas as pl
from jax.experimental.pallas import tpu as pltpu
from jax.experimental.pallas import tpu_sc as plsc

N = 10000
E = 320000
D = 128

NC = 2
NS = 16
NW = NC * NS
CHUNK = 128
CH = 79
PW = CH * CHUNK
EP = NW * PW
ACC_ROWS = 10240
DUMMY_COL = N

_mesh = plsc.VectorSubcoreMesh(core_axis_name="c", subcore_axis_name="s")

_ZROWS_PER_TILE = ACC_ROWS // NS
_OROWS_PER_TILE = N // NS


@functools.partial(
    pl.kernel,
    out_type=jax.ShapeDtypeStruct((NC, ACC_ROWS, D), jnp.float32),
    mesh=_mesh,
    scratch_types=[
        pltpu.VMEM((CH, CHUNK), jnp.int32),
        pltpu.VMEM((CH, CHUNK), jnp.int32),
        pltpu.VMEM((CHUNK, D), jnp.float32),
        pltpu.VMEM_SHARED((ACC_ROWS, D), jnp.float32),
        pltpu.SemaphoreType.DMA,
    ],
)
def _spmm_sc(g_hbm, rowp_hbm, colp_hbm, zeros_hbm, out_hbm,
             row_v, col_v, rows_v, acc_sh, sem):
    c = lax.axis_index("c")
    s = lax.axis_index("s")
    w = c * NS + s

    zbase = s * _ZROWS_PER_TILE
    pltpu.sync_copy(zeros_hbm.at[pl.ds(zbase, _ZROWS_PER_TILE)],
                    acc_sh.at[pl.ds(zbase, _ZROWS_PER_TILE)])

    pltpu.sync_copy(rowp_hbm.at[w], row_v)
    pltpu.sync_copy(colp_hbm.at[w], col_v)

    plsc.subcore_barrier()

    def body(j, carry):
        pltpu.async_copy(g_hbm.at[row_v.at[j]], rows_v, sem).wait()
        pltpu.sync_copy(rows_v, acc_sh.at[col_v.at[j]], add=True)
        return carry

    lax.fori_loop(0, CH, body, 0)

    plsc.subcore_barrier()

    pltpu.sync_copy(acc_sh.at[pl.ds(zbase, _ZROWS_PER_TILE)],
                    out_hbm.at[c, pl.ds(zbase, _ZROWS_PER_TILE)])


@functools.partial(
    pl.kernel,
    out_type=jax.ShapeDtypeStruct((NC, ACC_ROWS), jnp.float32),
    mesh=_mesh,
    scratch_types=[
        pltpu.VMEM((CH, CHUNK), jnp.int32),
        pltpu.VMEM((CHUNK,), jnp.float32),
        pltpu.VMEM_SHARED((ACC_ROWS,), jnp.float32),
        pltpu.SemaphoreType.DMA,
    ],
)
def _deg_sc(colp_hbm, ones_hbm, zeros1_hbm, out_hbm, col_v, ones_v, acc_sh, sem):
    c = lax.axis_index("c")
    s = lax.axis_index("s")
    w = c * NS + s

    zbase = s * _ZROWS_PER_TILE
    pltpu.sync_copy(zeros1_hbm.at[pl.ds(zbase, _ZROWS_PER_TILE)],
                    acc_sh.at[pl.ds(zbase, _ZROWS_PER_TILE)])
    pltpu.sync_copy(colp_hbm.at[w], col_v)
    pltpu.sync_copy(ones_hbm, ones_v)

    plsc.subcore_barrier()

    def body(j, carry):
        pltpu.sync_copy(ones_v, acc_sh.at[col_v.at[j]], add=True)
        return carry

    lax.fori_loop(0, CH, body, 0)

    plsc.subcore_barrier()

    pltpu.sync_copy(acc_sh.at[pl.ds(zbase, _ZROWS_PER_TILE)],
                    out_hbm.at[c, pl.ds(zbase, _ZROWS_PER_TILE)])


R = 1000
GRID = N // R


def _rowspec(d):
    return pl.BlockSpec((R, d), lambda i: (i, 0))


def _pairspec(d):
    return pl.BlockSpec((NC, R, d), lambda i: (0, i, 0))


def _fullspec(shape):
    nd = len(shape)
    return pl.BlockSpec(shape, lambda i, _n=nd: (0,) * _n)


def _colspec():
    return pl.BlockSpec((R, 1), lambda i: (i, 0))


def _matmul_t(a, w):
    return lax.dot_general(a, w, (((1,), (1,)), ((), ())),
                           preferred_element_type=jnp.float32)


def _tc_deg_body(degp, dis_o):
    deg = degp[0] + degp[1] + 1.0
    dis_o[...] = lax.rsqrt(deg)[:, None]


def _tc_deg(degp):
    return pl.pallas_call(
        _tc_deg_body,
        grid=(1,),
        in_specs=[_fullspec((NC, ACC_ROWS))],
        out_specs=_fullspec((ACC_ROWS, 1)),
        out_shape=jax.ShapeDtypeStruct((ACC_ROWS, 1), jnp.float32),
    )(degp)


def _tc_a_body(dis, x, w1, b1, xs_o, t0_o):
    xs_o[...] = x[...] * dis[...]
    t0_o[...] = _matmul_t(x[...], w1[...]) + b1[...][None, :]


def _tc_a(dis, x, w1, b1):
    return pl.pallas_call(
        _tc_a_body,
        grid=(GRID,),
        in_specs=[_colspec(), _rowspec(D),
                  _fullspec((D, D)), _fullspec((D,))],
        out_specs=[_rowspec(D), _rowspec(D)],
        out_shape=[jax.ShapeDtypeStruct((N, D), jnp.float32),
                   jax.ShapeDtypeStruct((N, D), jnp.float32)],
    )(dis, x, w1, b1)


def _tc_b_body(p1, dis, xs, w1, b1, t1_o, ys_o):
    y1 = (p1[0] + p1[1] + xs[...]) * dis[...]
    t1_o[...] = _matmul_t(y1, w1[...]) + b1[...][None, :]
    ys_o[...] = y1 * dis[...]


def _tc_b(p1, dis, xs, w1, b1):
    return pl.pallas_call(
        _tc_b_body,
        grid=(GRID,),
        in_specs=[_pairspec(D), _colspec(), _rowspec(D),
                  _fullspec((D, D)), _fullspec((D,))],
        out_specs=[_rowspec(D), _rowspec(D)],
        out_shape=[jax.ShapeDtypeStruct((N, D), jnp.float32),
                   jax.ShapeDtypeStruct((N, D), jnp.float32)],
    )(p1, dis, xs, w1, b1)


def _tc_c_body(p2, dis, ys, t0, t1, w12, b12, w20, b20, w21, b21, w22, b22,
               u0_o, z1s_o, z2s_o):
    disv = dis[...]
    y2 = (p2[0] + p2[1] + ys[...]) * disv
    t2 = _matmul_t(y2, w12[...]) + b12[...][None, :]
    h = jnp.maximum(jnp.concatenate([t0[...], t1[...], t2], axis=1), 0.0)
    u0_o[...] = _matmul_t(h, w20[...]) + b20[...][None, :]
    z1s_o[...] = (_matmul_t(h, w21[...]) + b21[...][None, :]) * disv
    z2s_o[...] = (_matmul_t(h, w22[...]) + b22[...][None, :]) * disv


def _tc_c(p2, dis, ys, t0, t1, w12, b12, w20, b20, w21, b21, w22, b22):
    return pl.pallas_call(
        _tc_c_body,
        grid=(GRID,),
        in_specs=[_pairspec(D), _colspec(), _rowspec(D), _rowspec(D),
                  _rowspec(D), _fullspec((D, D)), _fullspec((D,)),
                  _fullspec((D, 3 * D)), _fullspec((D,)),
                  _fullspec((D, 3 * D)), _fullspec((D,)),
                  _fullspec((D, 3 * D)), _fullspec((D,))],
        out_specs=[_rowspec(D), _rowspec(D), _rowspec(D)],
        out_shape=[jax.ShapeDtypeStruct((N, D), jnp.float32),
                   jax.ShapeDtypeStruct((N, D), jnp.float32),
                   jax.ShapeDtypeStruct((N, D), jnp.float32)],
    )(p2, dis, ys, t0, t1, w12, b12, w20, b20, w21, b21, w22, b22)


def _tc_d_body(p3, p4, dis, z1s, z2s, u1_o, vs_o):
    disv = dis[...]
    u1_o[...] = (p3[0] + p3[1] + z1s[...]) * disv
    vs_o[...] = (p4[0] + p4[1] + z2s[...]) * disv * disv


def _tc_d(p3, p4, dis, z1s, z2s):
    return pl.pallas_call(
        _tc_d_body,
        grid=(GRID,),
        in_specs=[_pairspec(D), _pairspec(D), _colspec(), _rowspec(D),
                  _rowspec(D)],
        out_specs=[_rowspec(D), _rowspec(D)],
        out_shape=[jax.ShapeDtypeStruct((N, D), jnp.float32),
                   jax.ShapeDtypeStruct((N, D), jnp.float32)],
    )(p3, p4, dis, z1s, z2s)


def _tc_e_body(p5, dis, vs, u0, u1, wf, bf, out_o):
    u2 = (p5[0] + p5[1] + vs[...]) * dis[...]
    wfm = wf[...]
    acc = _matmul_t(u0[...], wfm[:, 0:D])
    acc = acc + _matmul_t(u1[...], wfm[:, D:2 * D])
    acc = acc + _matmul_t(u2, wfm[:, 2 * D:3 * D])
    out_o[...] = acc + bf[...][None, :]


def _tc_e(p5, dis, vs, u0, u1, wf, bf):
    return pl.pallas_call(
        _tc_e_body,
        grid=(GRID,),
        in_specs=[_pairspec(D), _colspec(), _rowspec(D), _rowspec(D),
                  _rowspec(D), _fullspec((D, 3 * D)), _fullspec((D,))],
        out_specs=_rowspec(D),
        out_shape=jax.ShapeDtypeStruct((N, D), jnp.float32),
    )(p5, dis, vs, u0, u1, wf, bf)


def kernel(x, edge_index, W1_0, b1_0, W1_1, b1_1, W1_2, b1_2,
           W2_0, b2_0, W2_1, b2_1, W2_2, b2_2, Wf, bf):
    row = edge_index[0].astype(jnp.int32)
    col = edge_index[1].astype(jnp.int32)
    npad = EP - E
    rowp = jnp.concatenate([row, jnp.zeros((npad,), jnp.int32)])
    colp = jnp.concatenate([col, jnp.full((npad,), DUMMY_COL, jnp.int32)])
    rowp = rowp.reshape(NW, CH, CHUNK)
    colp = colp.reshape(NW, CH, CHUNK)

    zeros2 = jnp.zeros((ACC_ROWS, D), jnp.float32)
    zeros1 = jnp.zeros((ACC_ROWS,), jnp.float32)
    ones1 = jnp.ones((CHUNK,), jnp.float32)

    degp = _deg_sc(colp, ones1, zeros1)
    dis = _tc_deg(degp)
    xs, t0 = _tc_a(dis, x, W1_0, b1_0)
    p1 = _spmm_sc(xs, rowp, colp, zeros2)
    t1, ys = _tc_b(p1, dis, xs, W1_1, b1_1)
    p2 = _spmm_sc(ys, rowp, colp, zeros2)
    u0, z1s, z2s = _tc_c(p2, dis, ys, t0, t1, W1_2, b1_2,
                         W2_0, b2_0, W2_1, b2_1, W2_2, b2_2)
    p3 = _spmm_sc(z1s, rowp, colp, zeros2)
    p4 = _spmm_sc(z2s, rowp, colp, zeros2)
    u1, vs = _tc_d(p3, p4, dis, z1s, z2s)
    p5 = _spmm_sc(vs, rowp, colp, zeros2)
    out = _tc_e(p5, dis, vs, u0, u1, Wf, bf)
    return out
